# baseline (speedup 1.0000x reference)
import jax
import jax.numpy as jnp
from jax.experimental import pallas as pl

B, N, D_IN, HIDDEN = 4096, 50, 64, 128
NS = 5
JB = 2
BJ = B // JB


def _tok_kernel(x_ref, m_ref, w1_ref, w2_ref, b1_ref, b2_ref, o_ref):
    w = (w1_ref[...] + w2_ref[...]).astype(jnp.bfloat16)
    b = (b1_ref[...] + b2_ref[...]).astype(jnp.bfloat16)
    waug = jnp.concatenate([w, b], axis=0)
    i = pl.program_id(1)
    for k in range(NS):
        slab = x_ref[k]
        mrow = m_ref[pl.ds(i * NS + k, 1), :]
        aug = jnp.concatenate([slab, mrow], axis=0)
        aug_m = aug * mrow
        acc = jax.lax.dot_general(
            aug_m.astype(jnp.bfloat16), waug,
            dimension_numbers=(((0,), (0,)), ((), ())),
            preferred_element_type=jnp.float32,
        )
        o_ref[k] = acc


def kernel(x_feats, feats_masks, W1, b1, W2, b2):
    xt = jnp.transpose(x_feats, (1, 2, 0))
    mt = jnp.transpose(feats_masks, (1, 0)).astype(jnp.float32)
    b1r = b1.reshape(1, HIDDEN)
    b2r = b2.reshape(1, HIDDEN)

    out = pl.pallas_call(
        _tok_kernel,
        grid=(JB, N // NS),
        in_specs=[
            pl.BlockSpec((NS, D_IN, BJ), lambda j, i: (i, 0, j)),
            pl.BlockSpec((N, BJ), lambda j, i: (0, j)),
            pl.BlockSpec((D_IN, HIDDEN), lambda j, i: (0, 0)),
            pl.BlockSpec((D_IN, HIDDEN), lambda j, i: (0, 0)),
            pl.BlockSpec((1, HIDDEN), lambda j, i: (0, 0)),
            pl.BlockSpec((1, HIDDEN), lambda j, i: (0, 0)),
        ],
        out_specs=pl.BlockSpec((NS, BJ, HIDDEN), lambda j, i: (i, j, 0)),
        out_shape=jax.ShapeDtypeStruct((N, B, HIDDEN), jnp.float32),
    )(xt, mt, W1, W2, b1r, b2r)
    return jnp.transpose(out, (1, 0, 2))


# speedup vs baseline: 1.0698x; 1.0698x over previous
import jax
import jax.numpy as jnp
from jax.experimental import pallas as pl

B, N, D_IN, HIDDEN = 4096, 50, 64, 128
NS = 5
JB = 1
BJ = B // JB


def _tok_kernel(x_ref, m_ref, w1_ref, w2_ref, b1_ref, b2_ref, o_ref):
    w = (w1_ref[...] + w2_ref[...]).astype(jnp.bfloat16)
    b = (b1_ref[...] + b2_ref[...]).astype(jnp.bfloat16)
    waug = jnp.concatenate([w, b], axis=0)
    i = pl.program_id(1)
    for k in range(NS):
        slab = x_ref[k]
        mrow = m_ref[pl.ds(i * NS + k, 1), :]
        aug = jnp.concatenate([slab, mrow], axis=0)
        aug_m = aug * mrow
        acc = jax.lax.dot_general(
            aug_m.astype(jnp.bfloat16), waug,
            dimension_numbers=(((0,), (0,)), ((), ())),
            preferred_element_type=jnp.float32,
        )
        o_ref[k] = acc


def kernel(x_feats, feats_masks, W1, b1, W2, b2):
    xt = jnp.transpose(x_feats, (1, 2, 0))
    mt = jnp.transpose(feats_masks, (1, 0)).astype(jnp.float32)
    b1r = b1.reshape(1, HIDDEN)
    b2r = b2.reshape(1, HIDDEN)

    out = pl.pallas_call(
        _tok_kernel,
        grid=(JB, N // NS),
        in_specs=[
            pl.BlockSpec((NS, D_IN, BJ), lambda j, i: (i, 0, j)),
            pl.BlockSpec((N, BJ), lambda j, i: (0, j)),
            pl.BlockSpec((D_IN, HIDDEN), lambda j, i: (0, 0)),
            pl.BlockSpec((D_IN, HIDDEN), lambda j, i: (0, 0)),
            pl.BlockSpec((1, HIDDEN), lambda j, i: (0, 0)),
            pl.BlockSpec((1, HIDDEN), lambda j, i: (0, 0)),
        ],
        out_specs=pl.BlockSpec((NS, BJ, HIDDEN), lambda j, i: (i, j, 0)),
        out_shape=jax.ShapeDtypeStruct((N, B, HIDDEN), jnp.float32),
    )(xt, mt, W1, W2, b1r, b2r)
    return jnp.transpose(out, (1, 0, 2))
